# baseline (speedup 1.0000x reference)
import functools

import jax
import jax.numpy as jnp
from jax import lax
from jax.experimental import pallas as pl
from jax.experimental.pallas import tpu as pltpu
from jax.experimental.pallas import tpu_sc as plsc

VOCAB = 100000
D_W = 128
D_P = 32
D_OUT = D_W + D_P
B = 1024
S = 200
N = B * S

NC = 2
NS = 16
NW = NC * NS
CB = 128
JB = B // CB
NCHUNK = S * JB
CPW = NCHUNK // NW
NP = 4
L = 16

_mesh = plsc.VectorSubcoreMesh(core_axis_name="c", subcore_axis_name="s")


@functools.partial(
    pl.kernel,
    mesh=_mesh,
    compiler_params=pltpu.CompilerParams(needs_layout_passes=False,
                                         disable_bounds_checks=True),
    out_type=jax.ShapeDtypeStruct((S, D_OUT, B), jnp.float32),
    scratch_types=[
        pltpu.VMEM((CPW, CB), jnp.int32),
        pltpu.VMEM((CB, D_W), jnp.float32),
        pltpu.VMEM((CB, D_W), jnp.float32),
        pltpu.VMEM((D_W, CB), jnp.float32),
        pltpu.VMEM((D_W, CB), jnp.float32),
        pltpu.VMEM((NP, D_P, CB), jnp.float32),
        pltpu.SemaphoreType.DMA((2,)),
        pltpu.SemaphoreType.DMA((2,)),
        pltpu.SemaphoreType.DMA((NP,)),
        pltpu.SemaphoreType.DMA((NP,)),
    ],
)
def _emb_kernel(tok_hbm, post_hbm, w_hbm, out_hbm,
                idx2, rows0, rows1, trans0, trans1, posb,
                gsem, wsem, psem_in, psem_out):
    wid = lax.axis_index("s") * NC + lax.axis_index("c")
    c0 = wid * CPW
    rows = (rows0, rows1)
    trans = (trans0, trans1)

    pltpu.sync_copy(tok_hbm.at[wid], idx2)

    def coords(g):
        c = c0 + g
        s = c // JB
        b0 = (c % JB) * CB
        return s, b0

    def start_gather(g, b):
        pltpu.async_copy(w_hbm.at[idx2.at[g]], rows[b], gsem.at[b])

    def wait_gather(g, b):
        pltpu.make_async_copy(w_hbm.at[idx2.at[g]], rows[b],
                              gsem.at[b]).wait()

    def start_wwrite(g, b):
        s, b0 = coords(g)
        pltpu.async_copy(trans[b],
                         out_hbm.at[s, pl.ds(0, D_W), pl.ds(b0, CB)],
                         wsem.at[b])

    def wait_wwrite(b):
        pltpu.make_async_copy(trans[b],
                              out_hbm.at[0, pl.ds(0, D_W), pl.ds(0, CB)],
                              wsem.at[b]).wait()

    def start_pin(g, bp):
        s, b0 = coords(g)
        pltpu.async_copy(post_hbm.at[s, :, pl.ds(b0, CB)],
                         posb.at[bp], psem_in.at[bp])

    def wait_pin(g, bp):
        s, b0 = coords(g)
        pltpu.make_async_copy(post_hbm.at[s, :, pl.ds(b0, CB)],
                              posb.at[bp], psem_in.at[bp]).wait()

    def start_pout(g, bp):
        s, b0 = coords(g)
        pltpu.async_copy(posb.at[bp],
                         out_hbm.at[s, pl.ds(D_W, D_P), pl.ds(b0, CB)],
                         psem_out.at[bp])

    def wait_pout(bp):
        pltpu.make_async_copy(posb.at[bp],
                              out_hbm.at[0, pl.ds(D_W, D_P), pl.ds(0, CB)],
                              psem_out.at[bp]).wait()

    viota = lax.iota(jnp.int32, L)

    tvecs = [g * L + viota for g in range(CB // L)]

    def transpose_chunk(b):
        src, dst = rows[b], trans[b]

        NG = CB // L

        @plsc.parallel_loop(0, D_W * NG, step=1, unroll=16)
        def _(p):
            d = p // NG
            t0 = (p % NG) * L
            dvec = jnp.zeros((L,), jnp.int32) + d
            v = plsc.load_gather(src, [t0 + viota, dvec])
            dst[d, pl.ds(t0, L)] = v

    start_gather(0, 0)
    start_gather(1, 1)
    start_pin(0, 0)
    start_pin(1, 1)

    def chunk_body(t, b):
        g = 2 * t + b
        wait_gather(g, b)

        @pl.when(g >= 2)
        def _():
            wait_wwrite(b)

        transpose_chunk(b)
        start_wwrite(g, b)

        @pl.when(g + 2 < CPW)
        def _():
            start_gather(g + 2, b)

        bp = lax.rem(g, NP)
        wait_pin(g, bp)
        start_pout(g, bp)
        bp2 = lax.rem(g + 2, NP)

        @pl.when(jnp.logical_and(g + 2 < CPW, g >= 2))
        def _():
            wait_pout(bp2)

        @pl.when(g + 2 < CPW)
        def _():
            start_pin(g + 2, bp2)

    def it(t, carry):
        chunk_body(t, 0)
        chunk_body(t, 1)
        return carry

    lax.fori_loop(0, CPW // 2, it, 0)
    for b in range(2):
        wait_wwrite(b)
    for bp in range(NP):
        wait_pout(bp)


def kernel(token_ids, pos_onehot, W):
    tok3 = token_ids.T.astype(jnp.int32).reshape(NW, CPW, CB)
    pos_t = pos_onehot.transpose(1, 2, 0)
    out_t = _emb_kernel(tok3, pos_t, W)
    return out_t.transpose(2, 0, 1)


# speedup vs baseline: 1.2835x; 1.2835x over previous
import functools

import jax
import jax.numpy as jnp
from jax import lax
from jax.experimental import pallas as pl
from jax.experimental.pallas import tpu as pltpu
from jax.experimental.pallas import tpu_sc as plsc

VOCAB = 100000
D_W = 128
D_P = 32
D_OUT = D_W + D_P
B = 1024
S = 200
N = B * S

NC = 2
NS = 16
NW = NC * NS
CB = 128
JB = B // CB
NCHUNK = S * JB
CPW = NCHUNK // NW
NP = 4
L = 16

_mesh = plsc.VectorSubcoreMesh(core_axis_name="c", subcore_axis_name="s")


@functools.partial(
    pl.kernel,
    mesh=_mesh,
    compiler_params=pltpu.CompilerParams(needs_layout_passes=False,
                                         disable_bounds_checks=True),
    out_type=jax.ShapeDtypeStruct((S, D_OUT, B), jnp.float32),
    scratch_types=[
        pltpu.VMEM((CPW, CB), jnp.int32),
        pltpu.VMEM((CB, D_W), jnp.float32),
        pltpu.VMEM((CB, D_W), jnp.float32),
        pltpu.VMEM((D_W, CB), jnp.float32),
        pltpu.VMEM((D_W, CB), jnp.float32),
        pltpu.VMEM((NP, D_P, CB), jnp.float32),
        pltpu.SemaphoreType.DMA((2,)),
        pltpu.SemaphoreType.DMA((2,)),
        pltpu.SemaphoreType.DMA((NP,)),
        pltpu.SemaphoreType.DMA((NP,)),
    ],
)
def _emb_kernel(tok_hbm, post_hbm, w_hbm, out_hbm,
                idx2, rows0, rows1, trans0, trans1, posb,
                gsem, wsem, psem_in, psem_out):
    wid = lax.axis_index("s") * NC + lax.axis_index("c")
    c0 = wid * CPW
    rows = (rows0, rows1)
    trans = (trans0, trans1)

    pltpu.sync_copy(tok_hbm.at[wid], idx2)

    def coords(g):
        c = c0 + g
        s = c // JB
        b0 = (c % JB) * CB
        return s, b0

    def start_gather(g, b):
        pltpu.async_copy(w_hbm.at[idx2.at[g]], rows[b], gsem.at[b])

    def wait_gather(g, b):
        pltpu.make_async_copy(w_hbm.at[idx2.at[g]], rows[b],
                              gsem.at[b]).wait()

    def start_wwrite(g, b):
        s, b0 = coords(g)
        pltpu.async_copy(trans[b],
                         out_hbm.at[s, pl.ds(0, D_W), pl.ds(b0, CB)],
                         wsem.at[b])

    def wait_wwrite(b):
        pltpu.make_async_copy(trans[b],
                              out_hbm.at[0, pl.ds(0, D_W), pl.ds(0, CB)],
                              wsem.at[b]).wait()

    def start_pin(g, bp):
        s, b0 = coords(g)
        pltpu.async_copy(post_hbm.at[s, :, pl.ds(b0, CB)],
                         posb.at[bp], psem_in.at[bp])

    def wait_pin(g, bp):
        s, b0 = coords(g)
        pltpu.make_async_copy(post_hbm.at[s, :, pl.ds(b0, CB)],
                              posb.at[bp], psem_in.at[bp]).wait()

    def start_pout(g, bp):
        s, b0 = coords(g)
        pltpu.async_copy(posb.at[bp],
                         out_hbm.at[s, pl.ds(D_W, D_P), pl.ds(b0, CB)],
                         psem_out.at[bp])

    def wait_pout(bp):
        pltpu.make_async_copy(posb.at[bp],
                              out_hbm.at[0, pl.ds(D_W, D_P), pl.ds(0, CB)],
                              psem_out.at[bp]).wait()

    viota = lax.iota(jnp.int32, L)

    kvecs = [k * L + viota for k in range(D_W // L)]

    def transpose_chunk(b):
        src, dst = rows[b], trans[b]

        @plsc.parallel_loop(0, CB, step=1, unroll=8)
        def _(t):
            tvec = jnp.zeros((L,), jnp.int32) + t
            for k in range(D_W // L):
                v = src[t, pl.ds(k * L, L)]
                plsc.store_scatter(dst, [kvecs[k], tvec], v)

    start_gather(0, 0)
    start_gather(1, 1)
    start_pin(0, 0)
    start_pin(1, 1)

    def chunk_body(t, b):
        g = 2 * t + b
        wait_gather(g, b)

        @pl.when(g >= 2)
        def _():
            wait_wwrite(b)

        transpose_chunk(b)
        start_wwrite(g, b)

        @pl.when(g + 2 < CPW)
        def _():
            start_gather(g + 2, b)

        bp = lax.rem(g, NP)
        wait_pin(g, bp)
        start_pout(g, bp)
        bp2 = lax.rem(g + 2, NP)

        @pl.when(jnp.logical_and(g + 2 < CPW, g >= 2))
        def _():
            wait_pout(bp2)

        @pl.when(g + 2 < CPW)
        def _():
            start_pin(g + 2, bp2)

    def it(t, carry):
        chunk_body(t, 0)
        chunk_body(t, 1)
        return carry

    lax.fori_loop(0, CPW // 2, it, 0)
    for b in range(2):
        wait_wwrite(b)
    for bp in range(NP):
        wait_pout(bp)


def kernel(token_ids, pos_onehot, W):
    tok3 = token_ids.T.astype(jnp.int32).reshape(NW, CPW, CB)
    pos_t = pos_onehot.transpose(1, 2, 0)
    out_t = _emb_kernel(tok3, pos_t, W)
    return out_t.transpose(2, 0, 1)


# speedup vs baseline: 1.5030x; 1.1710x over previous
import functools

import jax
import jax.numpy as jnp
from jax import lax
from jax.experimental import pallas as pl
from jax.experimental.pallas import tpu as pltpu
from jax.experimental.pallas import tpu_sc as plsc

VOCAB = 100000
D_W = 128
D_P = 32
D_OUT = D_W + D_P
B = 1024
S = 200
N = B * S

NC = 2
NS = 16
NW = NC * NS
ROWS_PER_W = N // NW
CHUNK = 128
STEPS = ROWS_PER_W // CHUNK
NB = 3

_mesh = plsc.VectorSubcoreMesh(core_axis_name="c", subcore_axis_name="s")


@functools.partial(
    pl.kernel,
    mesh=_mesh,
    out_type=jax.ShapeDtypeStruct((N, D_OUT), jnp.float32),
    scratch_types=[
        pltpu.VMEM((STEPS, CHUNK), jnp.int32),
        pltpu.VMEM((NB, CHUNK, D_W), jnp.float32),
        pltpu.VMEM((NB, CHUNK, D_P), jnp.float32),
        pltpu.SemaphoreType.DMA((NB,)),
        pltpu.SemaphoreType.DMA((NB,)),
    ],
)
def _emb_kernel(tok_hbm, pos_hbm, w_hbm, out_hbm, idx2, rows, posb, gsem, wsem):
    wid = lax.axis_index("s") * NC + lax.axis_index("c")
    base = wid * ROWS_PER_W

    pltpu.sync_copy(tok_hbm.at[wid], idx2)

    def start_in(g, b):
        pltpu.async_copy(w_hbm.at[idx2.at[g]], rows.at[b], gsem.at[b])
        pltpu.async_copy(pos_hbm.at[pl.ds(base + g * CHUNK, CHUNK)],
                         posb.at[b], gsem.at[b])

    def wait_in(g, b):
        pltpu.make_async_copy(w_hbm.at[idx2.at[g]], rows.at[b],
                              gsem.at[b]).wait()
        pltpu.make_async_copy(pos_hbm.at[pl.ds(base + g * CHUNK, CHUNK)],
                              posb.at[b], gsem.at[b]).wait()

    def start_out(g, b):
        pltpu.async_copy(
            rows.at[b],
            out_hbm.at[pl.ds(base + g * CHUNK, CHUNK), pl.ds(0, D_W)],
            wsem.at[b])
        pltpu.async_copy(
            posb.at[b],
            out_hbm.at[pl.ds(base + g * CHUNK, CHUNK), pl.ds(D_W, D_P)],
            wsem.at[b])

    def wait_out(b):
        pltpu.make_async_copy(
            rows.at[b],
            out_hbm.at[pl.ds(base, CHUNK), pl.ds(0, D_W)],
            wsem.at[b]).wait()
        pltpu.make_async_copy(
            posb.at[b],
            out_hbm.at[pl.ds(base, CHUNK), pl.ds(D_W, D_P)],
            wsem.at[b]).wait()

    start_in(0, 0)
    start_in(1, 1)

    def it(g, carry):
        b = lax.rem(g, NB)
        wait_in(g, b)
        start_out(g, b)
        b2 = lax.rem(g + 2, NB)

        @pl.when(jnp.logical_and(g + 2 < STEPS, g >= NB - 2))
        def _():
            wait_out(b2)

        @pl.when(g + 2 < STEPS)
        def _():
            start_in(g + 2, b2)

        return carry

    lax.fori_loop(0, STEPS, it, 0)
    for b in range(NB):
        wait_out(b)


def kernel(token_ids, pos_onehot, W):
    tok = token_ids.reshape(NW, STEPS, CHUNK).astype(jnp.int32)
    pos = pos_onehot.reshape(N, D_P)
    out = _emb_kernel(tok, pos, W)
    return out.reshape(B, S, D_OUT)
